# Initial kernel scaffold; baseline (speedup 1.0000x reference)
import jax
import jax.numpy as jnp
from jax.experimental import pallas as pl


def kernel(pos, normals, batch, params):
    raise NotImplementedError("write your pallas kernel here")


# speedup vs baseline: 4.4882x; 4.4882x over previous
import functools

import jax
import jax.numpy as jnp
from jax import lax
from jax.experimental import pallas as pl
from jax.experimental.pallas import tpu as pltpu
from jax.experimental.pallas import tpu_sc as plsc

NG = 10
NPG = 1000
NPAD = 1024
NTOT = NG * NPAD
K = 20
KPAD = 24
EPS = 1e-5
T = 128
NC, NS = 2, 16
NW = NC * NS
E = K * NTOT
RW = E // NW
CH = 128

_INF = float("inf")


def _knn_node_body(x_ref, wa_ref, ba_ref, wd_ref, idx_ref, a_ref, d_ref, sc_ref):
    b = pl.program_id(0)
    x = x_ref[...]
    sq = jnp.sum(x * x, axis=1, keepdims=True)
    gram = lax.dot_general(x, x, (((1,), (1,)), ((), ())),
                           preferred_element_type=jnp.float32)
    rowid = lax.broadcasted_iota(jnp.int32, (NPAD, NPAD), 0)
    score = sq - 2.0 * gram
    score = jnp.where(rowid < NPG, score, _INF)
    sc_ref[...] = score
    for k in range(K):
        cur = sc_ref[...]
        m = jnp.min(cur, axis=0, keepdims=True)
        cand = jnp.where(cur == m, rowid, jnp.int32(NTOT))
        j = jnp.min(cand, axis=0, keepdims=True)
        idx_ref[k:k + 1, :] = j + b * NPAD
        sc_ref[...] = jnp.where(rowid == j, _INF, cur)
    a_ref[...] = (jnp.dot(x, wa_ref[...], preferred_element_type=jnp.float32)
                  + ba_ref[...])
    d_ref[...] = jnp.dot(x, wd_ref[...], preferred_element_type=jnp.float32)


def _knn_stage(x, wa, ba, wd):
    F = x.shape[1]
    C = wa.shape[1]
    return pl.pallas_call(
        _knn_node_body,
        grid=(NG,),
        in_specs=[
            pl.BlockSpec((NPAD, F), lambda b: (b, 0)),
            pl.BlockSpec((F, C), lambda b: (0, 0)),
            pl.BlockSpec((1, C), lambda b: (0, 0)),
            pl.BlockSpec((F, C), lambda b: (0, 0)),
        ],
        out_specs=[
            pl.BlockSpec((KPAD, NPAD), lambda b: (0, b)),
            pl.BlockSpec((NPAD, C), lambda b: (b, 0)),
            pl.BlockSpec((NPAD, C), lambda b: (b, 0)),
        ],
        out_shape=[
            jax.ShapeDtypeStruct((KPAD, NTOT), jnp.int32),
            jax.ShapeDtypeStruct((NTOT, C), jnp.float32),
            jax.ShapeDtypeStruct((NTOT, C), jnp.float32),
        ],
        scratch_shapes=[pltpu.VMEM((NPAD, NPAD), jnp.float32)],
    )(x, wa, ba, wd)


def _gather_rows(d, idx):
    C = d.shape[1]
    mesh = plsc.VectorSubcoreMesh(core_axis_name="c", subcore_axis_name="s",
                                  num_cores=NC, num_subcores=NS)

    @functools.partial(
        pl.kernel,
        out_type=jax.ShapeDtypeStruct((E, C), jnp.float32),
        mesh=mesh,
        scratch_types=[
            pltpu.VMEM((CH,), jnp.int32),
            pltpu.VMEM((CH, C), jnp.float32),
            pltpu.SemaphoreType.DMA,
        ],
    )
    def gk(d_hbm, idx_hbm, out_hbm, idx_v, row_v, sem):
        wid = lax.axis_index("s") * NC + lax.axis_index("c")
        base = wid * RW

        def step(i, carry):
            start = base + i * CH
            pltpu.sync_copy(idx_hbm.at[pl.ds(start, CH)], idx_v)
            pltpu.async_copy(d_hbm.at[idx_v], row_v, sem).wait()
            pltpu.sync_copy(row_v, out_hbm.at[pl.ds(start, CH)])
            return carry

        lax.fori_loop(0, RW // CH, step, 0)

    return gk(d, idx)


def _edge_body(a_ref, g_ref, w2_ref, b2_ref, wg_ref, bg_ref, wb_ref, bb_ref,
               o_ref):
    t = pl.program_id(1)
    a = a_ref[...]
    CP = a.shape[1]
    C = w2_ref.shape[1]
    g3 = g_ref[...]
    h1 = jnp.maximum(g3 + a[None, :, :], 0.0)
    h2 = jnp.maximum(
        jnp.dot(h1.reshape(K * T, CP), w2_ref[...],
                preferred_element_type=jnp.float32) + b2_ref[...], 0.0)
    xm = jnp.max(h2.reshape(K, T, C), axis=0)
    gam = jnp.dot(xm, wg_ref[...], preferred_element_type=jnp.float32) + bg_ref[...]
    bet = jnp.dot(xm, wb_ref[...], preferred_element_type=jnp.float32) + bb_ref[...]
    f = gam * xm + bet
    rows = t * T + lax.broadcasted_iota(jnp.int32, (T, 1), 0)
    o_ref[...] = jnp.where(rows < NPG, f, 0.0)


def _edge_stage(a, g3, w2, b2, wg, bg, wb, bb):
    CP = a.shape[1]
    C = w2.shape[1]
    nt = NPAD // T
    return pl.pallas_call(
        _edge_body,
        grid=(NG, nt),
        in_specs=[
            pl.BlockSpec((T, CP), lambda b, t: (b * (NPAD // T) + t, 0)),
            pl.BlockSpec((K, T, CP), lambda b, t: (0, b * (NPAD // T) + t, 0)),
            pl.BlockSpec((CP, C), lambda b, t: (0, 0)),
            pl.BlockSpec((1, C), lambda b, t: (0, 0)),
            pl.BlockSpec((C, C), lambda b, t: (0, 0)),
            pl.BlockSpec((1, C), lambda b, t: (0, 0)),
            pl.BlockSpec((C, C), lambda b, t: (0, 0)),
            pl.BlockSpec((1, C), lambda b, t: (0, 0)),
        ],
        out_specs=pl.BlockSpec((T, C), lambda b, t: (b * (NPAD // T) + t, 0)),
        out_shape=jax.ShapeDtypeStruct((NTOT, C), jnp.float32),
    )(a, g3, w2, b2, wg, bg, wb, bb)


def _head_body(x1_ref, x2_ref, x3_ref, x4_ref, w1_ref, b1_ref, w2_ref, b2_ref,
               w3_ref, b3_ref, o_ref):
    colv = lax.broadcasted_iota(jnp.int32, (16, NTOT), 1)
    rowb = lax.broadcasted_iota(jnp.int32, (16, NTOT), 0)
    sel = jnp.where((colv // NPAD == rowb) & (colv % NPAD < NPG),
                    jnp.float32(1.0 / NPG), 0.0)
    p1 = jnp.dot(sel, x1_ref[...], preferred_element_type=jnp.float32)
    p2 = jnp.dot(sel, x2_ref[...], preferred_element_type=jnp.float32)
    p3 = jnp.dot(sel, x3_ref[...], preferred_element_type=jnp.float32)
    p4 = jnp.dot(sel, x4_ref[...], preferred_element_type=jnp.float32)
    h = jnp.concatenate([p1, p2, p3, p4], axis=1)
    h = jnp.maximum(jnp.dot(h, w1_ref[...], preferred_element_type=jnp.float32)
                    + b1_ref[...], 0.0)
    h = jnp.maximum(jnp.dot(h, w2_ref[...], preferred_element_type=jnp.float32)
                    + b2_ref[...], 0.0)
    o_ref[...] = jnp.dot(h, w3_ref[...], preferred_element_type=jnp.float32) + b3_ref[...]


def _head_stage(xs, w1, b1, w2, b2, w3, b3):
    ins = list(xs) + [w1, b1, w2, b2, w3, b3]
    return pl.pallas_call(
        _head_body,
        out_shape=jax.ShapeDtypeStruct((16, 128), jnp.float32),
    )(*ins)


def _fold_bn(p):
    s = p['g'] / jnp.sqrt(1.0 + EPS)
    return s


def kernel(pos, normals, batch, params):
    del batch

    x = jnp.concatenate([pos, normals], axis=1).reshape(NG, NPG, 6)
    x = jnp.pad(x, ((0, 0), (0, NPAD - NPG), (0, 2))).reshape(NTOT, 8)

    feats = []
    for li in range(4):
        p1, p2 = params[f'conv{li + 1}']
        fl = params[f'film{li + 1}']
        F = p1['W'].shape[0] // 2
        s1 = _fold_bn(p1)
        wa = (p1['W'][:F] - p1['W'][F:]) * s1[None, :]
        wd = p1['W'][F:] * s1[None, :]
        ba = (p1['b'] * s1 + p1['beta'])[None, :]
        s2 = _fold_bn(p2)
        w2 = p2['W'] * s2[None, :]
        b2 = (p2['b'] * s2 + p2['beta'])[None, :]
        if li == 0:
            wa = jnp.pad(wa, ((0, 2), (0, 64)))
            wd = jnp.pad(wd, ((0, 2), (0, 64)))
            ba = jnp.pad(ba, ((0, 0), (0, 64)))
            w2 = jnp.pad(w2, ((0, 64), (0, 0)))

        idx, a, d = _knn_stage(x, wa, ba, wd)
        g = _gather_rows(d, idx[:K].reshape(E))
        C = d.shape[1]
        x = _edge_stage(a, g.reshape(K, NTOT, C), w2, b2,
                        fl['Wg'], fl['bg'][None], fl['Wb'], fl['bb'][None])
        feats.append(x)

    sh1 = params['bn1']['g'] / jnp.sqrt(1.0 + EPS)
    w1h = params['lin1']['W'] * sh1[None, :]
    b1h = (params['lin1']['b'] * sh1 + params['bn1']['beta'])[None, :]
    sh2 = params['bn2']['g'] / jnp.sqrt(1.0 + EPS)
    w2h = params['lin2']['W'] * sh2[None, :]
    b2h = (params['lin2']['b'] * sh2 + params['bn2']['beta'])[None, :]
    w3h = jnp.pad(params['lin3']['W'], ((0, 0), (0, 127)))
    b3h = jnp.pad(params['lin3']['b'], ((0, 127)))[None, :]

    out = _head_stage(feats, w1h, b1h, w2h, b2h, w3h, b3h)
    return out[:NG, :1]


# speedup vs baseline: 7.0689x; 1.5750x over previous
import functools

import jax
import jax.numpy as jnp
from jax import lax
from jax.experimental import pallas as pl
from jax.experimental.pallas import tpu as pltpu
from jax.experimental.pallas import tpu_sc as plsc

NG = 10
NPG = 1000
NPAD = 1024
NTOT = NG * NPAD
K = 20
KPAD = 24
EPS = 1e-5
T = 128
NC, NS = 2, 16
NW = NC * NS
E = K * NTOT
RW = E // NW
CH = 128

_INF = float("inf")


def _pack_bf16_pairs(v):
    C2 = v.shape[-1] // 2
    bl = lax.bitcast_convert_type(v[..., :C2], jnp.int32)
    bh = lax.bitcast_convert_type(v[..., C2:], jnp.int32)
    lo = ((bl + jnp.int32(0x8000)) >> 16) & jnp.int32(0xFFFF)
    hi = (bh + jnp.int32(0x8000)) & jnp.int32(-65536)
    return hi | lo


def _unpack_bf16_pairs(p):
    lo = lax.bitcast_convert_type(p << 16, jnp.float32)
    hi = lax.bitcast_convert_type(p & jnp.int32(-65536), jnp.float32)
    return jnp.concatenate([lo, hi], axis=-1)


def _knn_node_body(x_ref, wa_ref, ba_ref, wd_ref, idx_ref, a_ref, d_ref, sc_ref):
    b = pl.program_id(0)
    x = x_ref[...]
    sq = jnp.sum(x * x, axis=1, keepdims=True)
    gram = lax.dot_general(x, x, (((1,), (1,)), ((), ())),
                           preferred_element_type=jnp.float32)
    rowid = lax.broadcasted_iota(jnp.int32, (NPAD, NPAD), 0)
    score = sq - 2.0 * gram
    bits = lax.bitcast_convert_type(score, jnp.int32)
    keys = jnp.where(bits >= 0, bits, bits ^ jnp.int32(0x7FFFFFFF))
    keyq = (keys & jnp.int32(~1023)) | rowid
    keyq = jnp.where(rowid < NPG, keyq, jnp.int32(0x7FFFFFFF))
    sc_ref[...] = keyq
    for k in range(K):
        cur = sc_ref[...]
        m = jnp.min(cur, axis=0, keepdims=True)
        j = m & jnp.int32(1023)
        idx_ref[k:k + 1, :] = j + b * NPAD
        sc_ref[...] = jnp.where(rowid == j, jnp.int32(0x7FFFFFFF), cur)
    a_ref[...] = (jnp.dot(x, wa_ref[...], preferred_element_type=jnp.float32)
                  + ba_ref[...])
    dv = jnp.dot(x, wd_ref[...], preferred_element_type=jnp.float32)
    if d_ref.dtype == jnp.int32:
        d_ref[...] = _pack_bf16_pairs(dv)
    else:
        d_ref[...] = dv


def _knn_stage(x, wa, ba, wd, pack):
    F = x.shape[1]
    C = wa.shape[1]
    dcols, ddt = (C // 2, jnp.int32) if pack else (C, jnp.float32)
    return pl.pallas_call(
        _knn_node_body,
        grid=(NG,),
        in_specs=[
            pl.BlockSpec((NPAD, F), lambda b: (b, 0)),
            pl.BlockSpec((F, C), lambda b: (0, 0)),
            pl.BlockSpec((1, C), lambda b: (0, 0)),
            pl.BlockSpec((F, C), lambda b: (0, 0)),
        ],
        out_specs=[
            pl.BlockSpec((KPAD, NPAD), lambda b: (0, b)),
            pl.BlockSpec((NPAD, C), lambda b: (b, 0)),
            pl.BlockSpec((NPAD, dcols), lambda b: (b, 0)),
        ],
        out_shape=[
            jax.ShapeDtypeStruct((KPAD, NTOT), jnp.int32),
            jax.ShapeDtypeStruct((NTOT, C), jnp.float32),
            jax.ShapeDtypeStruct((NTOT, dcols), ddt),
        ],
        scratch_shapes=[pltpu.VMEM((NPAD, NPAD), jnp.int32)],
    )(x, wa, ba, wd)


NCHW = RW // CH


def _gather_rows(d, idx):
    C = d.shape[1]
    mesh = plsc.VectorSubcoreMesh(core_axis_name="c", subcore_axis_name="s",
                                  num_cores=NC, num_subcores=NS)

    @functools.partial(
        pl.kernel,
        out_type=jax.ShapeDtypeStruct((E, C), d.dtype),
        mesh=mesh,
        scratch_types=[
            pltpu.VMEM((RW,), jnp.int32),
            pltpu.VMEM((CH, C), d.dtype),
            pltpu.VMEM((CH, C), d.dtype),
            pltpu.SemaphoreType.DMA,
            pltpu.SemaphoreType.DMA,
            pltpu.SemaphoreType.DMA,
        ],
    )
    def gk(d_hbm, idx_hbm, out_hbm, idx_v, buf0, buf1, gs0, gs1, isem):
        wid = lax.axis_index("s") * NC + lax.axis_index("c")
        base = wid * RW
        pltpu.async_copy(idx_hbm.at[pl.ds(base, RW)], idx_v, isem).wait()
        bufs = (buf0, buf1)
        gsems = (gs0, gs1)

        def gstart(c, p):
            pltpu.async_copy(d_hbm.at[idx_v.at[pl.ds(c * CH, CH)]], bufs[p],
                             gsems[p])

        def gwait(c, p):
            pltpu.make_async_copy(d_hbm.at[idx_v.at[pl.ds(c * CH, CH)]],
                                  bufs[p], gsems[p]).wait()

        gstart(0, 0)

        def body(s, carry):
            for p in range(2):
                c = 2 * s + p

                @pl.when(c + 1 < NCHW)
                def _():
                    gstart(c + 1, 1 - p)

                gwait(c, p)
                pltpu.sync_copy(bufs[p], out_hbm.at[pl.ds(base + c * CH, CH)])
            return carry

        lax.fori_loop(0, NCHW // 2, body, 0)

    return gk(d, idx)


def _edge_body(a_ref, g_ref, w2_ref, b2_ref, wg_ref, bg_ref, wb_ref, bb_ref,
               o_ref):
    t = pl.program_id(1)
    a = a_ref[...]
    CP = a.shape[1]
    C = w2_ref.shape[1]
    if g_ref.dtype == jnp.int32:
        g3 = _unpack_bf16_pairs(g_ref[...])
    else:
        g3 = g_ref[...]
    h1 = jnp.maximum(g3 + a[None, :, :], 0.0).astype(jnp.bfloat16)
    h2 = jnp.maximum(
        jnp.dot(h1.reshape(K * T, CP), w2_ref[...],
                preferred_element_type=jnp.float32) + b2_ref[...], 0.0)
    xm = jnp.max(h2.reshape(K, T, C), axis=0)
    xb = xm.astype(jnp.bfloat16)
    gam = jnp.dot(xb, wg_ref[...], preferred_element_type=jnp.float32) + bg_ref[...]
    bet = jnp.dot(xb, wb_ref[...], preferred_element_type=jnp.float32) + bb_ref[...]
    f = gam * xm + bet
    rows = t * T + lax.broadcasted_iota(jnp.int32, (T, 1), 0)
    o_ref[...] = jnp.where(rows < NPG, f, 0.0)


def _edge_stage(a, g3, w2, b2, wg, bg, wb, bb):
    CP = a.shape[1]
    C = w2.shape[1]
    gc = g3.shape[2]
    nt = NPAD // T
    return pl.pallas_call(
        _edge_body,
        grid=(NG, nt),
        in_specs=[
            pl.BlockSpec((T, CP), lambda b, t: (b * (NPAD // T) + t, 0)),
            pl.BlockSpec((K, T, gc), lambda b, t: (0, b * (NPAD // T) + t, 0)),
            pl.BlockSpec((CP, C), lambda b, t: (0, 0)),
            pl.BlockSpec((1, C), lambda b, t: (0, 0)),
            pl.BlockSpec((C, C), lambda b, t: (0, 0)),
            pl.BlockSpec((1, C), lambda b, t: (0, 0)),
            pl.BlockSpec((C, C), lambda b, t: (0, 0)),
            pl.BlockSpec((1, C), lambda b, t: (0, 0)),
        ],
        out_specs=pl.BlockSpec((T, C), lambda b, t: (b * (NPAD // T) + t, 0)),
        out_shape=jax.ShapeDtypeStruct((NTOT, C), jnp.float32),
    )(a, g3, w2, b2, wg, bg, wb, bb)


def _head_body(x1_ref, x2_ref, x3_ref, x4_ref, w1_ref, b1_ref, w2_ref, b2_ref,
               w3_ref, b3_ref, o_ref):
    colv = lax.broadcasted_iota(jnp.int32, (16, NTOT), 1)
    rowb = lax.broadcasted_iota(jnp.int32, (16, NTOT), 0)
    sel = jnp.where((colv // NPAD == rowb) & (colv % NPAD < NPG),
                    jnp.float32(1.0 / NPG), 0.0)
    p1 = jnp.dot(sel, x1_ref[...], preferred_element_type=jnp.float32)
    p2 = jnp.dot(sel, x2_ref[...], preferred_element_type=jnp.float32)
    p3 = jnp.dot(sel, x3_ref[...], preferred_element_type=jnp.float32)
    p4 = jnp.dot(sel, x4_ref[...], preferred_element_type=jnp.float32)
    h = jnp.concatenate([p1, p2, p3, p4], axis=1)
    h = jnp.maximum(jnp.dot(h, w1_ref[...], preferred_element_type=jnp.float32)
                    + b1_ref[...], 0.0)
    h = jnp.maximum(jnp.dot(h, w2_ref[...], preferred_element_type=jnp.float32)
                    + b2_ref[...], 0.0)
    o_ref[...] = jnp.dot(h, w3_ref[...], preferred_element_type=jnp.float32) + b3_ref[...]


def _head_stage(xs, w1, b1, w2, b2, w3, b3):
    ins = list(xs) + [w1, b1, w2, b2, w3, b3]
    return pl.pallas_call(
        _head_body,
        out_shape=jax.ShapeDtypeStruct((16, 128), jnp.float32),
    )(*ins)


def _fold_bn(p):
    s = p['g'] / jnp.sqrt(1.0 + EPS)
    return s


def kernel(pos, normals, batch, params):
    del batch

    x = jnp.concatenate([pos, normals], axis=1).reshape(NG, NPG, 6)
    x = jnp.pad(x, ((0, 0), (0, NPAD - NPG), (0, 2))).reshape(NTOT, 8)

    feats = []
    for li in range(4):
        p1, p2 = params[f'conv{li + 1}']
        fl = params[f'film{li + 1}']
        F = p1['W'].shape[0] // 2
        s1 = _fold_bn(p1)
        wa = (p1['W'][:F] - p1['W'][F:]) * s1[None, :]
        wd = p1['W'][F:] * s1[None, :]
        ba = (p1['b'] * s1 + p1['beta'])[None, :]
        s2 = _fold_bn(p2)
        w2 = p2['W'] * s2[None, :]
        b2 = (p2['b'] * s2 + p2['beta'])[None, :]
        if li == 0:
            wa = jnp.pad(wa, ((0, 2), (0, 64)))
            wd = jnp.pad(wd, ((0, 2), (0, 64)))
            ba = jnp.pad(ba, ((0, 0), (0, 64)))
            w2 = jnp.pad(w2, ((0, 64), (0, 0)))

        C = wa.shape[1]
        pack = C >= 256
        idx, a, d = _knn_stage(x, wa, ba, wd, pack)
        g = _gather_rows(d, idx[:K].reshape(E))
        x = _edge_stage(a, g.reshape(K, NTOT, d.shape[1]),
                        w2.astype(jnp.bfloat16), b2,
                        fl['Wg'].astype(jnp.bfloat16), fl['bg'][None],
                        fl['Wb'].astype(jnp.bfloat16), fl['bb'][None])
        feats.append(x)

    sh1 = params['bn1']['g'] / jnp.sqrt(1.0 + EPS)
    w1h = params['lin1']['W'] * sh1[None, :]
    b1h = (params['lin1']['b'] * sh1 + params['bn1']['beta'])[None, :]
    sh2 = params['bn2']['g'] / jnp.sqrt(1.0 + EPS)
    w2h = params['lin2']['W'] * sh2[None, :]
    b2h = (params['lin2']['b'] * sh2 + params['bn2']['beta'])[None, :]
    w3h = jnp.pad(params['lin3']['W'], ((0, 0), (0, 127)))
    b3h = jnp.pad(params['lin3']['b'], ((0, 127)))[None, :]

    out = _head_stage(feats, w1h, b1h, w2h, b2h, w3h, b3h)
    return out[:NG, :1]
